# baseline (speedup 1.0000x reference)
import functools

import jax
import jax.numpy as jnp
from jax import lax
from jax.experimental import pallas as pl
from jax.experimental.pallas import tpu as pltpu
from jax.experimental.pallas import tpu_sc as plsc

B, Q, T, D, C = 64, 300, 1000, 256, 91

_FEATURE_DEN = float(B * Q * D)
_BOX_DEN = float(B * Q * 4)
_ROUTER_DEN = float(B * Q)

_NW = 32
_CH = 200


def _tc_body(srow_ref, brow_ref, side_ref, xt_ref, oboxt_ref, kl_ref, tr_ref,
             out_ref, idx_ref, acc_ref):
    b = pl.program_id(0)

    @pl.when(b == 0)
    def _init():
        for k in range(4):
            acc_ref[k] = 0.0

    r_row = srow_ref[0] + brow_ref[0]

    r_col = jnp.transpose(r_row, (1, 0))
    x_j = jnp.broadcast_to(r_col, (T, T))
    y_i = jnp.broadcast_to(r_row, (T, T))
    jlt = (lax.broadcasted_iota(jnp.int32, (T, T), 0)
           < lax.broadcasted_iota(jnp.int32, (T, T), 1))
    g = (jnp.where(x_j > y_i, 1.0, 0.0)
         + jnp.where((x_j == y_i) & jlt, 1.0, 0.0))
    rank = jnp.sum(g, axis=0, keepdims=True)

    rank_i = (rank + 0.5).astype(jnp.int32)
    prow = lax.broadcasted_iota(jnp.int32, (Q, T), 0)
    p_sel = prow == jnp.broadcast_to(rank_i, (Q, T))
    p_mat = jnp.where(p_sel, 1.0, 0.0)

    lane_i = lax.broadcasted_iota(jnp.int32, (Q, T), 1)
    idx_col = jnp.sum(jnp.where(p_sel, lane_i, 0), axis=1, keepdims=True)
    idx_row = jnp.transpose(idx_col, (1, 0)) + b * T
    idx_ref[0] = idx_row

    sel_t = lax.dot_general(side_ref[0], p_mat, (((1,), (1,)), ((), ())),
                            preferred_element_type=jnp.float32)
    bd = oboxt_ref[0] - sel_t[0:4, :]
    absd = jnp.abs(bd)
    sl1 = jnp.where(absd < 1.0, 0.5 * bd * bd, absd - 0.5)
    bsum_b = jnp.sum(sl1)

    xt = xt_ref[0]
    s_sum = jnp.sum(jnp.maximum(xt, 0.0) + jnp.log1p(jnp.exp(-jnp.abs(xt))),
                    axis=0, keepdims=True)
    alabel = (sel_t[5:6, :] + 0.5).astype(jnp.int32)
    onehot = lax.broadcasted_iota(jnp.int32, (C, Q), 0) == alabel
    xsel = jnp.sum(jnp.where(onehot, xt, 0.0), axis=0, keepdims=True)
    w = jnp.clip(sel_t[4:5, :], 0.0, 1.0)
    bce_b = jnp.sum(w * (s_sum - xsel))
    wsum_b = jnp.sum(w)

    rd = kl_ref[0] - tr_ref[0]
    rsum_b = jnp.sum(rd * rd)

    acc_ref[0] = acc_ref[0] + bsum_b
    acc_ref[1] = acc_ref[1] + rsum_b
    acc_ref[2] = acc_ref[2] + bce_b
    acc_ref[3] = acc_ref[3] + wsum_b

    @pl.when(b == B - 1)
    def _final():
        lane = lax.broadcasted_iota(jnp.int32, (8, 128), 1)
        row = lax.broadcasted_iota(jnp.int32, (8, 128), 0)
        out = (jnp.where((row == 0) & (lane == 0), acc_ref[0], 0.0)
               + jnp.where((row == 0) & (lane == 1), acc_ref[1], 0.0)
               + jnp.where((row == 0) & (lane == 2), acc_ref[2], 0.0)
               + jnp.where((row == 0) & (lane == 3), acc_ref[3], 0.0))
        out_ref[...] = out


def _sc_body(feat_hbm, q_hbm, idx_hbm, out_hbm, idx_v, f_v, q_v, acc_v, sem):
    wid = lax.axis_index("s") * 2 + lax.axis_index("c")
    acc_v[...] = jnp.zeros((16,), jnp.float32)
    base = wid * (2 * Q)
    for c in range(3):
        pltpu.sync_copy(idx_hbm.at[pl.ds(base + c * _CH, _CH)], idx_v)
        pltpu.async_copy(feat_hbm.at[idx_v], f_v, sem).wait()
        pltpu.sync_copy(q_hbm.at[pl.ds(base + c * _CH, _CH)], q_v)

        def row_body(r, carry):
            s = jnp.zeros((16,), jnp.float32)
            for dv in range(D // 16):
                sl = pl.ds(dv * 16, 16)
                dq = q_v[r, sl] - f_v[r, sl]
                s = s + dq * dq
            acc_v[...] = acc_v[...] + s
            return carry

        lax.fori_loop(0, _CH, row_body, 0)
    pltpu.sync_copy(acc_v, out_hbm.at[pl.ds(wid * 16, 16)])


def _comb_body(l_ref, p_ref, o_ref):
    lane = lax.broadcasted_iota(jnp.int32, (8, 128), 1)
    row = lax.broadcasted_iota(jnp.int32, (8, 128), 0)
    vals = l_ref[...]
    fsum = jnp.sum(p_ref[...])
    bsum = jnp.sum(jnp.where((row == 0) & (lane == 0), vals, 0.0))
    rsum = jnp.sum(jnp.where((row == 0) & (lane == 1), vals, 0.0))
    bce = jnp.sum(jnp.where((row == 0) & (lane == 2), vals, 0.0))
    wsum = jnp.sum(jnp.where((row == 0) & (lane == 3), vals, 0.0))
    feature_loss = fsum / _FEATURE_DEN
    box_loss = bsum / _BOX_DEN
    router_loss = rsum / _ROUTER_DEN * 0.5
    logits_loss = 0.5 * bce / jnp.maximum(float(C) * wsum, 1.0)
    total = feature_loss + box_loss + router_loss + logits_loss
    out = (jnp.where((row == 0) & (lane == 0), total, 0.0)
           + jnp.where((row == 0) & (lane == 1), feature_loss, 0.0)
           + jnp.where((row == 0) & (lane == 2), box_loss, 0.0)
           + jnp.where((row == 0) & (lane == 3), router_loss, 0.0)
           + jnp.where((row == 0) & (lane == 4), logits_loss, 0.0))
    o_ref[...] = out


def kernel(object_logits, object_queries, object_boxes, seed_bank_keep_logits,
           teacher_object_features, teacher_object_boxes, teacher_object_labels,
           teacher_object_scores, teacher_router_logits, teacher_valid_mask):
    del teacher_valid_mask

    f32 = jnp.float32
    bias = jnp.linspace(0.0, -1e-06 * (T - 1), T).astype(f32)
    scores = teacher_object_scores.astype(f32)
    srow = scores.reshape(B, 1, T)
    brow = bias.reshape(1, 1, T)

    side = jnp.concatenate([
        jnp.moveaxis(teacher_object_boxes.astype(f32), 2, 1),
        scores[:, None, :],
        teacher_object_labels.astype(f32)[:, None, :],
    ], axis=1)
    xt = jnp.moveaxis(object_logits, 2, 1)
    oboxt = jnp.moveaxis(object_boxes.astype(f32), 2, 1)

    kl = seed_bank_keep_logits.reshape(B, 1, Q)
    tr = teacher_router_logits.reshape(B, 1, Q)

    losses, idx = pl.pallas_call(
        _tc_body,
        grid=(B,),
        in_specs=[
            pl.BlockSpec((1, 1, T), lambda b: (b, 0, 0)),
            pl.BlockSpec((1, 1, T), lambda b: (0, 0, 0)),
            pl.BlockSpec((1, 6, T), lambda b: (b, 0, 0)),
            pl.BlockSpec((1, C, Q), lambda b: (b, 0, 0)),
            pl.BlockSpec((1, 4, Q), lambda b: (b, 0, 0)),
            pl.BlockSpec((1, 1, Q), lambda b: (b, 0, 0)),
            pl.BlockSpec((1, 1, Q), lambda b: (b, 0, 0)),
        ],
        out_specs=[
            pl.BlockSpec((8, 128), lambda b: (0, 0)),
            pl.BlockSpec((1, 1, Q), lambda b: (b, 0, 0)),
        ],
        out_shape=[
            jax.ShapeDtypeStruct((8, 128), f32),
            jax.ShapeDtypeStruct((B, 1, Q), jnp.int32),
        ],
        scratch_shapes=[pltpu.SMEM((8,), f32)],
    )(srow, brow, side, xt, oboxt, kl, tr)

    feat2d = teacher_object_features.reshape(B * T, D)
    q2d = object_queries.reshape(B * Q, D)
    idx1d = idx.reshape(B * Q)

    mesh = plsc.VectorSubcoreMesh(core_axis_name="c", subcore_axis_name="s")
    sc_call = functools.partial(
        pl.kernel, mesh=mesh,
        out_type=jax.ShapeDtypeStruct((_NW * 16,), f32),
        scratch_types=[
            pltpu.VMEM((_CH,), jnp.int32),
            pltpu.VMEM((_CH, D), f32),
            pltpu.VMEM((_CH, D), f32),
            pltpu.VMEM((16,), f32),
            pltpu.SemaphoreType.DMA,
        ],
    )(_sc_body)
    partials = sc_call(feat2d, q2d, idx1d)

    out = pl.pallas_call(
        _comb_body,
        in_specs=[
            pl.BlockSpec((8, 128), lambda: (0, 0)),
            pl.BlockSpec((4, 128), lambda: (0, 0)),
        ],
        out_specs=pl.BlockSpec((8, 128), lambda: (0, 0)),
        out_shape=jax.ShapeDtypeStruct((8, 128), f32),
    )(losses, partials.reshape(4, 128))
    return out[0, :5]


# speedup vs baseline: 1.0047x; 1.0047x over previous
import functools

import jax
import jax.numpy as jnp
from jax import lax
from jax.experimental import pallas as pl
from jax.experimental.pallas import tpu as pltpu
from jax.experimental.pallas import tpu_sc as plsc

B, Q, T, D, C = 64, 300, 1000, 256, 91

_FEATURE_DEN = float(B * Q * D)
_BOX_DEN = float(B * Q * 4)
_ROUTER_DEN = float(B * Q)

_NW = 32
_CH = 152
_CHUNKS = ((0, 152), (152, 152), (304, 152), (456, 144))


def _tc_body(srow_ref, brow_ref, side_ref, xt_ref, oboxt_ref, kl_ref, tr_ref,
             out_ref, idx_ref, acc_ref):
    b = pl.program_id(0)

    @pl.when(b == 0)
    def _init():
        for k in range(4):
            acc_ref[k] = 0.0

    r_row = srow_ref[0] + brow_ref[0]

    r_col = jnp.transpose(r_row, (1, 0))
    x_j = jnp.broadcast_to(r_col, (T, T))
    y_i = jnp.broadcast_to(r_row, (T, T))
    jlt = (lax.broadcasted_iota(jnp.int32, (T, T), 0)
           < lax.broadcasted_iota(jnp.int32, (T, T), 1))
    g = (jnp.where(x_j > y_i, 1.0, 0.0)
         + jnp.where((x_j == y_i) & jlt, 1.0, 0.0))
    rank = jnp.sum(g, axis=0, keepdims=True)

    rank_i = (rank + 0.5).astype(jnp.int32)
    prow = lax.broadcasted_iota(jnp.int32, (Q, T), 0)
    p_sel = prow == jnp.broadcast_to(rank_i, (Q, T))
    p_mat = jnp.where(p_sel, 1.0, 0.0)

    lane_i = lax.broadcasted_iota(jnp.int32, (Q, T), 1)
    idx_col = jnp.sum(jnp.where(p_sel, lane_i, 0), axis=1, keepdims=True)
    idx_row = jnp.transpose(idx_col, (1, 0)) + b * T
    idx_ref[0] = idx_row

    sel_t = lax.dot_general(side_ref[0], p_mat, (((1,), (1,)), ((), ())),
                            preferred_element_type=jnp.float32)
    bd = oboxt_ref[0] - sel_t[0:4, :]
    absd = jnp.abs(bd)
    sl1 = jnp.where(absd < 1.0, 0.5 * bd * bd, absd - 0.5)
    bsum_b = jnp.sum(sl1)

    xt = xt_ref[0]
    s_sum = jnp.sum(jnp.maximum(xt, 0.0) + jnp.log1p(jnp.exp(-jnp.abs(xt))),
                    axis=0, keepdims=True)
    alabel = (sel_t[5:6, :] + 0.5).astype(jnp.int32)
    onehot = lax.broadcasted_iota(jnp.int32, (C, Q), 0) == alabel
    xsel = jnp.sum(jnp.where(onehot, xt, 0.0), axis=0, keepdims=True)
    w = jnp.clip(sel_t[4:5, :], 0.0, 1.0)
    bce_b = jnp.sum(w * (s_sum - xsel))
    wsum_b = jnp.sum(w)

    rd = kl_ref[0] - tr_ref[0]
    rsum_b = jnp.sum(rd * rd)

    acc_ref[0] = acc_ref[0] + bsum_b
    acc_ref[1] = acc_ref[1] + rsum_b
    acc_ref[2] = acc_ref[2] + bce_b
    acc_ref[3] = acc_ref[3] + wsum_b

    @pl.when(b == B - 1)
    def _final():
        lane = lax.broadcasted_iota(jnp.int32, (8, 128), 1)
        row = lax.broadcasted_iota(jnp.int32, (8, 128), 0)
        out = (jnp.where((row == 0) & (lane == 0), acc_ref[0], 0.0)
               + jnp.where((row == 0) & (lane == 1), acc_ref[1], 0.0)
               + jnp.where((row == 0) & (lane == 2), acc_ref[2], 0.0)
               + jnp.where((row == 0) & (lane == 3), acc_ref[3], 0.0))
        out_ref[...] = out


def _sc_body(feat_hbm, q_hbm, idx_hbm, out_hbm, idx_v, f0_v, f1_v, q_v,
             acc_v, sem0, sem1):
    wid = lax.axis_index("s") * 2 + lax.axis_index("c")
    acc_v[...] = jnp.zeros((16,), jnp.float32)
    base = wid * (2 * Q)
    pltpu.sync_copy(idx_hbm.at[pl.ds(base, 2 * Q)], idx_v)

    bufs = (f0_v, f1_v)
    sems = (sem0, sem1)
    handles = [None] * len(_CHUNKS)
    off0, len0 = _CHUNKS[0]
    handles[0] = pltpu.async_copy(
        feat_hbm.at[idx_v.at[pl.ds(off0, len0)]], bufs[0], sems[0])
    for c, (off, ln) in enumerate(_CHUNKS):
        if c + 1 < len(_CHUNKS):
            noff, nln = _CHUNKS[c + 1]
            nb = bufs[(c + 1) % 2]
            handles[c + 1] = pltpu.async_copy(
                feat_hbm.at[idx_v.at[pl.ds(noff, nln)]],
                nb.at[pl.ds(0, nln)], sems[(c + 1) % 2])
        pltpu.sync_copy(q_hbm.at[pl.ds(base + off, ln)], q_v.at[pl.ds(0, ln)])
        handles[c].wait()
        fb = bufs[c % 2]

        def row_body(r, carry):
            s = jnp.zeros((16,), jnp.float32)
            for dv in range(D // 16):
                sl = pl.ds(dv * 16, 16)
                dq = q_v[r, sl] - fb[r, sl]
                s = s + dq * dq
            acc_v[...] = acc_v[...] + s
            return carry

        lax.fori_loop(0, ln, row_body, 0)
    pltpu.sync_copy(acc_v, out_hbm.at[pl.ds(wid * 16, 16)])


def _comb_body(l_ref, p_ref, o_ref):
    lane = lax.broadcasted_iota(jnp.int32, (8, 128), 1)
    row = lax.broadcasted_iota(jnp.int32, (8, 128), 0)
    vals = l_ref[...]
    fsum = jnp.sum(p_ref[...])
    bsum = jnp.sum(jnp.where((row == 0) & (lane == 0), vals, 0.0))
    rsum = jnp.sum(jnp.where((row == 0) & (lane == 1), vals, 0.0))
    bce = jnp.sum(jnp.where((row == 0) & (lane == 2), vals, 0.0))
    wsum = jnp.sum(jnp.where((row == 0) & (lane == 3), vals, 0.0))
    feature_loss = fsum / _FEATURE_DEN
    box_loss = bsum / _BOX_DEN
    router_loss = rsum / _ROUTER_DEN * 0.5
    logits_loss = 0.5 * bce / jnp.maximum(float(C) * wsum, 1.0)
    total = feature_loss + box_loss + router_loss + logits_loss
    out = (jnp.where((row == 0) & (lane == 0), total, 0.0)
           + jnp.where((row == 0) & (lane == 1), feature_loss, 0.0)
           + jnp.where((row == 0) & (lane == 2), box_loss, 0.0)
           + jnp.where((row == 0) & (lane == 3), router_loss, 0.0)
           + jnp.where((row == 0) & (lane == 4), logits_loss, 0.0))
    o_ref[...] = out


def kernel(object_logits, object_queries, object_boxes, seed_bank_keep_logits,
           teacher_object_features, teacher_object_boxes, teacher_object_labels,
           teacher_object_scores, teacher_router_logits, teacher_valid_mask):
    del teacher_valid_mask

    f32 = jnp.float32
    bias = jnp.linspace(0.0, -1e-06 * (T - 1), T).astype(f32)
    scores = teacher_object_scores.astype(f32)
    srow = scores.reshape(B, 1, T)
    brow = bias.reshape(1, 1, T)

    side = jnp.concatenate([
        jnp.moveaxis(teacher_object_boxes.astype(f32), 2, 1),
        scores[:, None, :],
        teacher_object_labels.astype(f32)[:, None, :],
    ], axis=1)
    xt = jnp.moveaxis(object_logits, 2, 1)
    oboxt = jnp.moveaxis(object_boxes.astype(f32), 2, 1)

    kl = seed_bank_keep_logits.reshape(B, 1, Q)
    tr = teacher_router_logits.reshape(B, 1, Q)

    losses, idx = pl.pallas_call(
        _tc_body,
        grid=(B,),
        in_specs=[
            pl.BlockSpec((1, 1, T), lambda b: (b, 0, 0)),
            pl.BlockSpec((1, 1, T), lambda b: (0, 0, 0)),
            pl.BlockSpec((1, 6, T), lambda b: (b, 0, 0)),
            pl.BlockSpec((1, C, Q), lambda b: (b, 0, 0)),
            pl.BlockSpec((1, 4, Q), lambda b: (b, 0, 0)),
            pl.BlockSpec((1, 1, Q), lambda b: (b, 0, 0)),
            pl.BlockSpec((1, 1, Q), lambda b: (b, 0, 0)),
        ],
        out_specs=[
            pl.BlockSpec((8, 128), lambda b: (0, 0)),
            pl.BlockSpec((1, 1, Q), lambda b: (b, 0, 0)),
        ],
        out_shape=[
            jax.ShapeDtypeStruct((8, 128), f32),
            jax.ShapeDtypeStruct((B, 1, Q), jnp.int32),
        ],
        scratch_shapes=[pltpu.SMEM((8,), f32)],
    )(srow, brow, side, xt, oboxt, kl, tr)

    feat2d = teacher_object_features.reshape(B * T, D)
    q2d = object_queries.reshape(B * Q, D)
    idx1d = idx.reshape(B * Q)

    mesh = plsc.VectorSubcoreMesh(core_axis_name="c", subcore_axis_name="s")
    sc_call = functools.partial(
        pl.kernel, mesh=mesh,
        out_type=jax.ShapeDtypeStruct((_NW * 16,), f32),
        scratch_types=[
            pltpu.VMEM((2 * Q,), jnp.int32),
            pltpu.VMEM((_CH, D), f32),
            pltpu.VMEM((_CH, D), f32),
            pltpu.VMEM((_CH, D), f32),
            pltpu.VMEM((16,), f32),
            pltpu.SemaphoreType.DMA,
            pltpu.SemaphoreType.DMA,
        ],
    )(_sc_body)
    partials = sc_call(feat2d, q2d, idx1d)

    out = pl.pallas_call(
        _comb_body,
        in_specs=[
            pl.BlockSpec((8, 128), lambda: (0, 0)),
            pl.BlockSpec((4, 128), lambda: (0, 0)),
        ],
        out_specs=pl.BlockSpec((8, 128), lambda: (0, 0)),
        out_shape=jax.ShapeDtypeStruct((8, 128), f32),
    )(losses, partials.reshape(4, 128))
    return out[0, :5]


# speedup vs baseline: 1.0178x; 1.0131x over previous
import functools

import jax
import jax.numpy as jnp
from jax import lax
from jax.experimental import pallas as pl
from jax.experimental.pallas import tpu as pltpu
from jax.experimental.pallas import tpu_sc as plsc

B, Q, T, D, C = 64, 300, 1000, 256, 91

_FEATURE_DEN = float(B * Q * D)
_BOX_DEN = float(B * Q * 4)
_ROUTER_DEN = float(B * Q)

_NW = 32
_CH = 152
_CHUNKS = ((0, 152), (152, 152), (304, 152), (456, 144))


def _tc_body(srow_ref, brow_ref, side_ref, xt_ref, oboxt_ref, kl_ref, tr_ref,
             out_ref, idx_ref, acc_ref):
    b = pl.program_id(0)

    @pl.when(b == 0)
    def _init():
        for k in range(4):
            acc_ref[k] = 0.0

    r_row = srow_ref[0] + brow_ref[0]

    r_col = jnp.transpose(r_row, (1, 0))
    x_j = jnp.broadcast_to(r_col, (T, T))
    y_i = jnp.broadcast_to(r_row, (T, T))
    jlt = (lax.broadcasted_iota(jnp.int32, (T, T), 0)
           < lax.broadcasted_iota(jnp.int32, (T, T), 1))
    g = (jnp.where(x_j > y_i, 1.0, 0.0)
         + jnp.where((x_j == y_i) & jlt, 1.0, 0.0))
    rank = jnp.sum(g, axis=0, keepdims=True)

    rank_i = (rank + 0.5).astype(jnp.int32)
    prow = lax.broadcasted_iota(jnp.int32, (Q, T), 0)
    p_sel = prow == jnp.broadcast_to(rank_i, (Q, T))
    p_mat = jnp.where(p_sel, 1.0, 0.0)

    lane_i = lax.broadcasted_iota(jnp.int32, (Q, T), 1)
    idx_col = jnp.sum(jnp.where(p_sel, lane_i, 0), axis=1, keepdims=True)
    idx_row = jnp.transpose(idx_col, (1, 0)) + b * T
    idx_ref[0] = idx_row

    sel_t = lax.dot_general(side_ref[0], p_mat, (((1,), (1,)), ((), ())),
                            preferred_element_type=jnp.float32)
    bd = oboxt_ref[0] - sel_t[0:4, :]
    absd = jnp.abs(bd)
    sl1 = jnp.where(absd < 1.0, 0.5 * bd * bd, absd - 0.5)
    bsum_b = jnp.sum(sl1)

    xt = xt_ref[0]
    s_sum = jnp.sum(jnp.maximum(xt, 0.0) + jnp.log1p(jnp.exp(-jnp.abs(xt))),
                    axis=0, keepdims=True)
    alabel = (sel_t[5:6, :] + 0.5).astype(jnp.int32)
    onehot = lax.broadcasted_iota(jnp.int32, (C, Q), 0) == alabel
    xsel = jnp.sum(jnp.where(onehot, xt, 0.0), axis=0, keepdims=True)
    w = jnp.clip(sel_t[4:5, :], 0.0, 1.0)
    bce_b = jnp.sum(w * (s_sum - xsel))
    wsum_b = jnp.sum(w)

    rd = kl_ref[0] - tr_ref[0]
    rsum_b = jnp.sum(rd * rd)

    acc_ref[0] = acc_ref[0] + bsum_b
    acc_ref[1] = acc_ref[1] + rsum_b
    acc_ref[2] = acc_ref[2] + bce_b
    acc_ref[3] = acc_ref[3] + wsum_b

    @pl.when(b == B - 1)
    def _final():
        lane = lax.broadcasted_iota(jnp.int32, (8, 128), 1)
        row = lax.broadcasted_iota(jnp.int32, (8, 128), 0)
        out = (jnp.where((row == 0) & (lane == 0), acc_ref[0], 0.0)
               + jnp.where((row == 0) & (lane == 1), acc_ref[1], 0.0)
               + jnp.where((row == 0) & (lane == 2), acc_ref[2], 0.0)
               + jnp.where((row == 0) & (lane == 3), acc_ref[3], 0.0))
        out_ref[...] = out


def _sc_body(feat_hbm, q_hbm, idx_hbm, out_hbm, idx_v, f0_v, f1_v, q_v,
             acc_v, sem0, sem1):
    wid = lax.axis_index("s") * 2 + lax.axis_index("c")
    acc_v[...] = jnp.zeros((16,), jnp.float32)
    base = wid * (2 * Q)
    pltpu.sync_copy(idx_hbm.at[pl.ds(base, 2 * Q)], idx_v)

    bufs = (f0_v, f1_v)
    sems = (sem0, sem1)
    handles = [None] * len(_CHUNKS)
    off0, len0 = _CHUNKS[0]
    handles[0] = pltpu.async_copy(
        feat_hbm.at[idx_v.at[pl.ds(off0, len0)]], bufs[0], sems[0])
    for c, (off, ln) in enumerate(_CHUNKS):
        if c + 1 < len(_CHUNKS):
            noff, nln = _CHUNKS[c + 1]
            nb = bufs[(c + 1) % 2]
            handles[c + 1] = pltpu.async_copy(
                feat_hbm.at[idx_v.at[pl.ds(noff, nln)]],
                nb.at[pl.ds(0, nln)], sems[(c + 1) % 2])
        pltpu.sync_copy(q_hbm.at[pl.ds(base + off, ln)], q_v.at[pl.ds(0, ln)])
        handles[c].wait()
        fb = bufs[c % 2]

        def row_body(r2, carry):
            s = jnp.zeros((16,), jnp.float32)
            for rr in range(2):
                for dv in range(D // 16):
                    sl = pl.ds(dv * 16, 16)
                    dq = q_v[2 * r2 + rr, sl] - fb[2 * r2 + rr, sl]
                    s = s + dq * dq
            acc_v[...] = acc_v[...] + s
            return carry

        lax.fori_loop(0, ln // 2, row_body, 0)
    pltpu.sync_copy(acc_v, out_hbm.at[pl.ds(wid * 16, 16)])


def _comb_body(l_ref, p_ref, o_ref):
    lane = lax.broadcasted_iota(jnp.int32, (8, 128), 1)
    row = lax.broadcasted_iota(jnp.int32, (8, 128), 0)
    vals = l_ref[...]
    fsum = jnp.sum(p_ref[...])
    bsum = jnp.sum(jnp.where((row == 0) & (lane == 0), vals, 0.0))
    rsum = jnp.sum(jnp.where((row == 0) & (lane == 1), vals, 0.0))
    bce = jnp.sum(jnp.where((row == 0) & (lane == 2), vals, 0.0))
    wsum = jnp.sum(jnp.where((row == 0) & (lane == 3), vals, 0.0))
    feature_loss = fsum / _FEATURE_DEN
    box_loss = bsum / _BOX_DEN
    router_loss = rsum / _ROUTER_DEN * 0.5
    logits_loss = 0.5 * bce / jnp.maximum(float(C) * wsum, 1.0)
    total = feature_loss + box_loss + router_loss + logits_loss
    out = (jnp.where((row == 0) & (lane == 0), total, 0.0)
           + jnp.where((row == 0) & (lane == 1), feature_loss, 0.0)
           + jnp.where((row == 0) & (lane == 2), box_loss, 0.0)
           + jnp.where((row == 0) & (lane == 3), router_loss, 0.0)
           + jnp.where((row == 0) & (lane == 4), logits_loss, 0.0))
    o_ref[...] = out


def kernel(object_logits, object_queries, object_boxes, seed_bank_keep_logits,
           teacher_object_features, teacher_object_boxes, teacher_object_labels,
           teacher_object_scores, teacher_router_logits, teacher_valid_mask):
    del teacher_valid_mask

    f32 = jnp.float32
    bias = jnp.linspace(0.0, -1e-06 * (T - 1), T).astype(f32)
    scores = teacher_object_scores.astype(f32)
    srow = scores.reshape(B, 1, T)
    brow = bias.reshape(1, 1, T)

    side = jnp.concatenate([
        jnp.moveaxis(teacher_object_boxes.astype(f32), 2, 1),
        scores[:, None, :],
        teacher_object_labels.astype(f32)[:, None, :],
    ], axis=1)
    xt = jnp.moveaxis(object_logits, 2, 1)
    oboxt = jnp.moveaxis(object_boxes.astype(f32), 2, 1)

    kl = seed_bank_keep_logits.reshape(B, 1, Q)
    tr = teacher_router_logits.reshape(B, 1, Q)

    losses, idx = pl.pallas_call(
        _tc_body,
        grid=(B,),
        in_specs=[
            pl.BlockSpec((1, 1, T), lambda b: (b, 0, 0)),
            pl.BlockSpec((1, 1, T), lambda b: (0, 0, 0)),
            pl.BlockSpec((1, 6, T), lambda b: (b, 0, 0)),
            pl.BlockSpec((1, C, Q), lambda b: (b, 0, 0)),
            pl.BlockSpec((1, 4, Q), lambda b: (b, 0, 0)),
            pl.BlockSpec((1, 1, Q), lambda b: (b, 0, 0)),
            pl.BlockSpec((1, 1, Q), lambda b: (b, 0, 0)),
        ],
        out_specs=[
            pl.BlockSpec((8, 128), lambda b: (0, 0)),
            pl.BlockSpec((1, 1, Q), lambda b: (b, 0, 0)),
        ],
        out_shape=[
            jax.ShapeDtypeStruct((8, 128), f32),
            jax.ShapeDtypeStruct((B, 1, Q), jnp.int32),
        ],
        scratch_shapes=[pltpu.SMEM((8,), f32)],
    )(srow, brow, side, xt, oboxt, kl, tr)

    feat2d = teacher_object_features.reshape(B * T, D)
    q2d = object_queries.reshape(B * Q, D)
    idx1d = idx.reshape(B * Q)

    mesh = plsc.VectorSubcoreMesh(core_axis_name="c", subcore_axis_name="s")
    sc_call = functools.partial(
        pl.kernel, mesh=mesh,
        out_type=jax.ShapeDtypeStruct((_NW * 16,), f32),
        scratch_types=[
            pltpu.VMEM((2 * Q,), jnp.int32),
            pltpu.VMEM((_CH, D), f32),
            pltpu.VMEM((_CH, D), f32),
            pltpu.VMEM((_CH, D), f32),
            pltpu.VMEM((16,), f32),
            pltpu.SemaphoreType.DMA,
            pltpu.SemaphoreType.DMA,
        ],
    )(_sc_body)
    partials = sc_call(feat2d, q2d, idx1d)

    out = pl.pallas_call(
        _comb_body,
        in_specs=[
            pl.BlockSpec((8, 128), lambda: (0, 0)),
            pl.BlockSpec((4, 128), lambda: (0, 0)),
        ],
        out_specs=pl.BlockSpec((8, 128), lambda: (0, 0)),
        out_shape=jax.ShapeDtypeStruct((8, 128), f32),
    )(losses, partials.reshape(4, 128))
    return out[0, :5]


# speedup vs baseline: 1.0535x; 1.0351x over previous
import functools

import jax
import jax.numpy as jnp
from jax import lax
from jax.experimental import pallas as pl
from jax.experimental.pallas import tpu as pltpu
from jax.experimental.pallas import tpu_sc as plsc

B, Q, T, D, C = 64, 300, 1000, 256, 91

_FEATURE_DEN = float(B * Q * D)
_BOX_DEN = float(B * Q * 4)
_ROUTER_DEN = float(B * Q)

_NW = 32
_CH = 152
_CHUNKS = ((0, 152), (152, 152), (304, 152), (456, 144))


def _rank_body(srow_ref, brow_ref, rank_ref, idx_ref):
    b = pl.program_id(0)
    r_row = srow_ref[0] + brow_ref[0]

    r_col = jnp.transpose(r_row, (1, 0))
    x_j = jnp.broadcast_to(r_col, (T, T))
    y_i = jnp.broadcast_to(r_row, (T, T))
    jlt = (lax.broadcasted_iota(jnp.int32, (T, T), 0)
           < lax.broadcasted_iota(jnp.int32, (T, T), 1))
    g = (jnp.where(x_j > y_i, 1.0, 0.0)
         + jnp.where((x_j == y_i) & jlt, 1.0, 0.0))
    rank = jnp.sum(g, axis=0, keepdims=True)

    rank_i = (rank + 0.5).astype(jnp.int32)
    rank_ref[0] = rank_i
    p_sel = (lax.broadcasted_iota(jnp.int32, (Q, T), 0)
             == jnp.broadcast_to(rank_i, (Q, T)))
    lane_i = lax.broadcasted_iota(jnp.int32, (Q, T), 1)
    idx_col = jnp.sum(jnp.where(p_sel, lane_i, 0), axis=1, keepdims=True)
    idx_row = jnp.transpose(idx_col, (1, 0)) + b * T
    idx_ref[0] = idx_row


def _loss_body(rank_ref, side_ref, xt_ref, oboxt_ref, kl_ref, tr_ref,
               out_ref, acc_ref):
    b = pl.program_id(0)

    @pl.when(b == 0)
    def _init():
        for k in range(4):
            acc_ref[k] = 0.0

    rank_i = rank_ref[0]
    prow = lax.broadcasted_iota(jnp.int32, (Q, T), 0)
    p_sel = prow == jnp.broadcast_to(rank_i, (Q, T))
    p_mat = jnp.where(p_sel, 1.0, 0.0)

    sel_t = lax.dot_general(side_ref[0], p_mat, (((1,), (1,)), ((), ())),
                            preferred_element_type=jnp.float32)
    bd = oboxt_ref[0] - sel_t[0:4, :]
    absd = jnp.abs(bd)
    sl1 = jnp.where(absd < 1.0, 0.5 * bd * bd, absd - 0.5)
    bsum_b = jnp.sum(sl1)

    xt = xt_ref[0]
    s_sum = jnp.sum(jnp.maximum(xt, 0.0) + jnp.log1p(jnp.exp(-jnp.abs(xt))),
                    axis=0, keepdims=True)
    alabel = (sel_t[5:6, :] + 0.5).astype(jnp.int32)
    onehot = lax.broadcasted_iota(jnp.int32, (C, Q), 0) == alabel
    xsel = jnp.sum(jnp.where(onehot, xt, 0.0), axis=0, keepdims=True)
    w = jnp.clip(sel_t[4:5, :], 0.0, 1.0)
    bce_b = jnp.sum(w * (s_sum - xsel))
    wsum_b = jnp.sum(w)

    rd = kl_ref[0] - tr_ref[0]
    rsum_b = jnp.sum(rd * rd)

    acc_ref[0] = acc_ref[0] + bsum_b
    acc_ref[1] = acc_ref[1] + rsum_b
    acc_ref[2] = acc_ref[2] + bce_b
    acc_ref[3] = acc_ref[3] + wsum_b

    @pl.when(b == B - 1)
    def _final():
        lane = lax.broadcasted_iota(jnp.int32, (8, 128), 1)
        row = lax.broadcasted_iota(jnp.int32, (8, 128), 0)
        out = (jnp.where((row == 0) & (lane == 0), acc_ref[0], 0.0)
               + jnp.where((row == 0) & (lane == 1), acc_ref[1], 0.0)
               + jnp.where((row == 0) & (lane == 2), acc_ref[2], 0.0)
               + jnp.where((row == 0) & (lane == 3), acc_ref[3], 0.0))
        out_ref[...] = out


def _sc_body(feat_hbm, q_hbm, idx_hbm, out_hbm, idx_v, f0_v, f1_v, q_v,
             acc_v, sem0, sem1):
    wid = lax.axis_index("s") * 2 + lax.axis_index("c")
    acc_v[...] = jnp.zeros((16,), jnp.float32)
    base = wid * (2 * Q)
    pltpu.sync_copy(idx_hbm.at[pl.ds(base, 2 * Q)], idx_v)

    bufs = (f0_v, f1_v)
    sems = (sem0, sem1)
    handles = [None] * len(_CHUNKS)
    off0, len0 = _CHUNKS[0]
    handles[0] = pltpu.async_copy(
        feat_hbm.at[idx_v.at[pl.ds(off0, len0)]], bufs[0], sems[0])
    for c, (off, ln) in enumerate(_CHUNKS):
        if c + 1 < len(_CHUNKS):
            noff, nln = _CHUNKS[c + 1]
            nb = bufs[(c + 1) % 2]
            handles[c + 1] = pltpu.async_copy(
                feat_hbm.at[idx_v.at[pl.ds(noff, nln)]],
                nb.at[pl.ds(0, nln)], sems[(c + 1) % 2])
        pltpu.sync_copy(q_hbm.at[pl.ds(base + off, ln)], q_v.at[pl.ds(0, ln)])
        handles[c].wait()
        fb = bufs[c % 2]

        def row_body(r2, carry):
            s = jnp.zeros((16,), jnp.float32)
            for rr in range(2):
                for dv in range(D // 16):
                    sl = pl.ds(dv * 16, 16)
                    dq = q_v[2 * r2 + rr, sl] - fb[2 * r2 + rr, sl]
                    s = s + dq * dq
            acc_v[...] = acc_v[...] + s
            return carry

        lax.fori_loop(0, ln // 2, row_body, 0)
    pltpu.sync_copy(acc_v, out_hbm.at[pl.ds(wid * 16, 16)])


def _comb_body(l_ref, p_ref, o_ref):
    lane = lax.broadcasted_iota(jnp.int32, (8, 128), 1)
    row = lax.broadcasted_iota(jnp.int32, (8, 128), 0)
    vals = l_ref[...]
    fsum = jnp.sum(p_ref[...])
    bsum = jnp.sum(jnp.where((row == 0) & (lane == 0), vals, 0.0))
    rsum = jnp.sum(jnp.where((row == 0) & (lane == 1), vals, 0.0))
    bce = jnp.sum(jnp.where((row == 0) & (lane == 2), vals, 0.0))
    wsum = jnp.sum(jnp.where((row == 0) & (lane == 3), vals, 0.0))
    feature_loss = fsum / _FEATURE_DEN
    box_loss = bsum / _BOX_DEN
    router_loss = rsum / _ROUTER_DEN * 0.5
    logits_loss = 0.5 * bce / jnp.maximum(float(C) * wsum, 1.0)
    total = feature_loss + box_loss + router_loss + logits_loss
    out = (jnp.where((row == 0) & (lane == 0), total, 0.0)
           + jnp.where((row == 0) & (lane == 1), feature_loss, 0.0)
           + jnp.where((row == 0) & (lane == 2), box_loss, 0.0)
           + jnp.where((row == 0) & (lane == 3), router_loss, 0.0)
           + jnp.where((row == 0) & (lane == 4), logits_loss, 0.0))
    o_ref[...] = out


def kernel(object_logits, object_queries, object_boxes, seed_bank_keep_logits,
           teacher_object_features, teacher_object_boxes, teacher_object_labels,
           teacher_object_scores, teacher_router_logits, teacher_valid_mask):
    del teacher_valid_mask

    f32 = jnp.float32
    bias = jnp.linspace(0.0, -1e-06 * (T - 1), T).astype(f32)
    scores = teacher_object_scores.astype(f32)
    srow = scores.reshape(B, 1, T)
    brow = bias.reshape(1, 1, T)

    side = jnp.concatenate([
        jnp.moveaxis(teacher_object_boxes.astype(f32), 2, 1),
        scores[:, None, :],
        teacher_object_labels.astype(f32)[:, None, :],
    ], axis=1)
    xt = jnp.moveaxis(object_logits, 2, 1)
    oboxt = jnp.moveaxis(object_boxes.astype(f32), 2, 1)

    kl = seed_bank_keep_logits.reshape(B, 1, Q)
    tr = teacher_router_logits.reshape(B, 1, Q)

    rank, idx = pl.pallas_call(
        _rank_body,
        grid=(B,),
        in_specs=[
            pl.BlockSpec((1, 1, T), lambda b: (b, 0, 0)),
            pl.BlockSpec((1, 1, T), lambda b: (0, 0, 0)),
        ],
        out_specs=[
            pl.BlockSpec((1, 1, T), lambda b: (b, 0, 0)),
            pl.BlockSpec((1, 1, Q), lambda b: (b, 0, 0)),
        ],
        out_shape=[
            jax.ShapeDtypeStruct((B, 1, T), jnp.int32),
            jax.ShapeDtypeStruct((B, 1, Q), jnp.int32),
        ],
    )(srow, brow)

    losses = pl.pallas_call(
        _loss_body,
        grid=(B,),
        in_specs=[
            pl.BlockSpec((1, 1, T), lambda b: (b, 0, 0)),
            pl.BlockSpec((1, 6, T), lambda b: (b, 0, 0)),
            pl.BlockSpec((1, C, Q), lambda b: (b, 0, 0)),
            pl.BlockSpec((1, 4, Q), lambda b: (b, 0, 0)),
            pl.BlockSpec((1, 1, Q), lambda b: (b, 0, 0)),
            pl.BlockSpec((1, 1, Q), lambda b: (b, 0, 0)),
        ],
        out_specs=pl.BlockSpec((8, 128), lambda b: (0, 0)),
        out_shape=jax.ShapeDtypeStruct((8, 128), f32),
        scratch_shapes=[pltpu.SMEM((8,), f32)],
    )(rank, side, xt, oboxt, kl, tr)

    feat2d = teacher_object_features.reshape(B * T, D)
    q2d = object_queries.reshape(B * Q, D)
    idx1d = idx.reshape(B * Q)

    mesh = plsc.VectorSubcoreMesh(core_axis_name="c", subcore_axis_name="s")
    sc_call = functools.partial(
        pl.kernel, mesh=mesh,
        out_type=jax.ShapeDtypeStruct((_NW * 16,), f32),
        scratch_types=[
            pltpu.VMEM((2 * Q,), jnp.int32),
            pltpu.VMEM((_CH, D), f32),
            pltpu.VMEM((_CH, D), f32),
            pltpu.VMEM((_CH, D), f32),
            pltpu.VMEM((16,), f32),
            pltpu.SemaphoreType.DMA,
            pltpu.SemaphoreType.DMA,
        ],
    )(_sc_body)
    partials = sc_call(feat2d, q2d, idx1d)

    out = pl.pallas_call(
        _comb_body,
        in_specs=[
            pl.BlockSpec((8, 128), lambda: (0, 0)),
            pl.BlockSpec((4, 128), lambda: (0, 0)),
        ],
        out_specs=pl.BlockSpec((8, 128), lambda: (0, 0)),
        out_shape=jax.ShapeDtypeStruct((8, 128), f32),
    )(losses, partials.reshape(4, 128))
    return out[0, :5]
